# baseline (speedup 1.0000x reference)
import functools

import jax
import jax.numpy as jnp
from jax import lax
from jax.experimental import pallas as pl
from jax.experimental.pallas import tpu as pltpu
from jax.experimental.pallas import tpu_sc as plsc

NUM_CLASSES = 1000
N_ROWS = 4096
N_COLS = 26
N_ENTRIES = N_ROWS * N_COLS
N_WORKERS = 32
ENTRIES_PER_WORKER = N_ENTRIES // N_WORKERS
CHUNK = 32
N_CHUNKS = ENTRIES_PER_WORKER // CHUNK


def _sc_body(xf_hbm, out_hbm, xv, buf0, buf1, sem0, sem1):
    wid = lax.axis_index("s") * 2 + lax.axis_index("c")
    ii = lax.iota(jnp.int32, 16)
    zeros16 = jnp.zeros((16,), jnp.float32)
    ones16 = jnp.ones((16,), jnp.float32)

    pltpu.sync_copy(
        xf_hbm.at[pl.ds(wid * ENTRIES_PER_WORKER, ENTRIES_PER_WORKER)],
        xv.at[pl.ds(0, ENTRIES_PER_WORKER)],
    )

    def memset(buf):
        def row_body(r, c):
            def m_body(m, cc):
                off = pl.multiple_of(m * 16, 16)
                buf[r, pl.ds(off, 16)] = zeros16
                return cc

            lax.fori_loop(0, 62, m_body, c)
            rv = jnp.full((16,), r, jnp.int32)
            plsc.store_scatter(buf, [rv, 992 + ii], zeros16, mask=ii < 8)
            return c

        lax.fori_loop(0, CHUNK, row_body, 0)

    memset(buf0)
    memset(buf1)

    def paint(buf, k, value16):
        base = k * CHUNK
        for i in range(2):
            l = ii + (i * 16)
            cls = plsc.load_gather(xv, [base + i * 16 + ii])
            plsc.store_scatter(buf, [l, cls], value16)

    def copy_op(buf, k, sem):
        dst = out_hbm.at[pl.ds(wid * ENTRIES_PER_WORKER + k * CHUNK, CHUNK)]
        return pltpu.make_async_copy(buf, dst, sem)

    paint(buf0, 0, ones16)
    copy_op(buf0, 0, sem0).start()
    paint(buf1, 1, ones16)
    copy_op(buf1, 1, sem1).start()

    def chunk_step(buf, sem, k):
        copy_op(buf, k - 2, sem).wait()
        paint(buf, k - 2, zeros16)
        paint(buf, k, ones16)
        copy_op(buf, k, sem).start()

    def loop_body(g, c):
        chunk_step(buf0, sem0, 2 * g)
        chunk_step(buf1, sem1, 2 * g + 1)
        return c

    lax.fori_loop(1, N_CHUNKS // 2, loop_body, 0)

    copy_op(buf0, N_CHUNKS - 2, sem0).wait()
    copy_op(buf1, N_CHUNKS - 1, sem1).wait()


def kernel(X):
    xf = jnp.reshape(X, (-1,)).astype(jnp.int32)
    run = functools.partial(
        pl.kernel,
        out_type=jax.ShapeDtypeStruct((N_ENTRIES, NUM_CLASSES), jnp.float32),
        mesh=plsc.VectorSubcoreMesh(core_axis_name="c", subcore_axis_name="s"),
        scratch_types=[
            pltpu.VMEM((ENTRIES_PER_WORKER + 16,), jnp.int32),
            pltpu.VMEM((CHUNK, NUM_CLASSES), jnp.float32),
            pltpu.VMEM((CHUNK, NUM_CLASSES), jnp.float32),
            pltpu.SemaphoreType.DMA,
            pltpu.SemaphoreType.DMA,
        ],
        compiler_params=pltpu.CompilerParams(
            needs_layout_passes=False, use_tc_tiling_on_sc=True
        ),
    )(_sc_body)
    out2d = run(xf)
    return jnp.reshape(out2d, (N_ROWS, N_COLS, NUM_CLASSES))


# speedup vs baseline: 1.3444x; 1.3444x over previous
import functools

import jax
import jax.numpy as jnp
from jax import lax
from jax.experimental import pallas as pl
from jax.experimental.pallas import tpu as pltpu
from jax.experimental.pallas import tpu_sc as plsc

NUM_CLASSES = 1000
N_ROWS = 4096
N_COLS = 26
N_WORKERS = 32
ROWS_PER_WORKER = N_ROWS // N_WORKERS
ENTRIES_PER_WORKER = ROWS_PER_WORKER * N_COLS
CHUNK_ROWS = 1
CHUNK_ENTRIES = CHUNK_ROWS * N_COLS
N_CHUNKS = ROWS_PER_WORKER // CHUNK_ROWS


def _sc_body(xf_hbm, out_hbm, xv, buf0, buf1, sem0, sem1):
    wid = lax.axis_index("s") * 2 + lax.axis_index("c")
    ii = lax.iota(jnp.int32, 16)
    zeros16 = jnp.zeros((16,), jnp.float32)
    ones16 = jnp.ones((16,), jnp.float32)

    pltpu.sync_copy(
        xf_hbm.at[pl.ds(wid * ENTRIES_PER_WORKER, ENTRIES_PER_WORKER)],
        xv.at[pl.ds(0, ENTRIES_PER_WORKER)],
    )

    def memset(buf):
        def row_body(r, c):
            a = r // N_COLS
            b = r - a * N_COLS

            def m_body(m, cc):
                off = pl.multiple_of(m * 16, 16)
                buf[a, b, pl.ds(off, 16)] = zeros16
                return cc

            lax.fori_loop(0, 62, m_body, c)
            av = jnp.full((16,), a, jnp.int32)
            bv = jnp.full((16,), b, jnp.int32)
            plsc.store_scatter(buf, [av, bv, 992 + ii], zeros16, mask=ii < 8)
            return c

        lax.fori_loop(0, CHUNK_ROWS * N_COLS, row_body, 0)

    memset(buf0)
    memset(buf1)

    def paint(buf, k, value16):
        base = k * CHUNK_ENTRIES
        for i in range(2):
            l = ii + (i * 16)
            cls = plsc.load_gather(xv, [base + i * 16 + ii])
            row = l // N_COLS
            col = l - row * N_COLS
            plsc.store_scatter(buf, [row, col, cls], value16, mask=l < CHUNK_ENTRIES)

    def copy_op(buf, k, sem):
        dst = out_hbm.at[pl.ds(wid * ROWS_PER_WORKER + k * CHUNK_ROWS, CHUNK_ROWS)]
        return pltpu.make_async_copy(buf, dst, sem)

    paint(buf0, 0, ones16)
    copy_op(buf0, 0, sem0).start()
    paint(buf1, 1, ones16)
    copy_op(buf1, 1, sem1).start()

    def chunk_step(buf, sem, k):
        copy_op(buf, k - 2, sem).wait()
        paint(buf, k - 2, zeros16)
        paint(buf, k, ones16)
        copy_op(buf, k, sem).start()

    def loop_body(g, c):
        chunk_step(buf0, sem0, 2 * g)
        chunk_step(buf1, sem1, 2 * g + 1)
        return c

    lax.fori_loop(1, N_CHUNKS // 2, loop_body, 0)

    copy_op(buf0, N_CHUNKS - 2, sem0).wait()
    copy_op(buf1, N_CHUNKS - 1, sem1).wait()


def kernel(X):
    xf = jnp.reshape(X, (-1,)).astype(jnp.int32)
    run = functools.partial(
        pl.kernel,
        out_type=jax.ShapeDtypeStruct((N_ROWS, N_COLS, NUM_CLASSES), jnp.float32),
        mesh=plsc.VectorSubcoreMesh(core_axis_name="c", subcore_axis_name="s"),
        scratch_types=[
            pltpu.VMEM((ENTRIES_PER_WORKER + 16,), jnp.int32),
            pltpu.VMEM((CHUNK_ROWS, N_COLS, NUM_CLASSES), jnp.float32),
            pltpu.VMEM((CHUNK_ROWS, N_COLS, NUM_CLASSES), jnp.float32),
            pltpu.SemaphoreType.DMA,
            pltpu.SemaphoreType.DMA,
        ],
        compiler_params=pltpu.CompilerParams(
            needs_layout_passes=False, use_tc_tiling_on_sc=True
        ),
    )(_sc_body)
    return run(xf)
